# Initial kernel scaffold; baseline (speedup 1.0000x reference)
import jax
import jax.numpy as jnp
from jax.experimental import pallas as pl


def kernel(input_batch, indices, W, b):
    raise NotImplementedError("write your pallas kernel here")


# speedup vs baseline: 4.1415x; 4.1415x over previous
import functools

import jax
import jax.numpy as jnp
from jax import lax
from jax.experimental import pallas as pl
from jax.experimental.pallas import tpu as pltpu
from jax.experimental.pallas import tpu_sc as plsc

_NC = 2
_NS = 16
_NW = _NC * _NS

_BM = 128


def _sc_gather_rows(table, idx, n, d):
    b_per_w = n // _NW
    mesh = plsc.VectorSubcoreMesh(core_axis_name="c", subcore_axis_name="s")

    @functools.partial(
        pl.kernel,
        mesh=mesh,
        out_type=jax.ShapeDtypeStruct((n, d), jnp.float32),
        scratch_types=[
            pltpu.VMEM((b_per_w,), jnp.int32),
            pltpu.VMEM((b_per_w, d), jnp.float32),
            pltpu.SemaphoreType.DMA,
        ],
    )
    def gather_kernel(table_hbm, idx_hbm, out_hbm, idx_v, rows_v, sem):
        wid = lax.axis_index("s") * _NC + lax.axis_index("c")
        base = wid * b_per_w
        pltpu.sync_copy(idx_hbm.at[pl.ds(base, b_per_w)], idx_v)
        pltpu.async_copy(table_hbm.at[idx_v], rows_v, sem).wait()
        pltpu.sync_copy(rows_v, out_hbm.at[pl.ds(base, b_per_w)])

    return gather_kernel(table, idx)


def _gmm_kernel(gid_ref, blk_ref, lo_ref, hi_ref, x_ref, w_ref, b_ref, o_ref):
    s = pl.program_id(0)
    lo = lo_ref[s]
    hi = hi_ref[s]
    blk = blk_ref[s]
    bm = o_ref.shape[0]
    rows = blk * bm + lax.broadcasted_iota(jnp.int32, (bm, 1), 0)
    mask = (rows >= lo) & (rows < hi)
    acc = jnp.dot(x_ref[...], w_ref[0], preferred_element_type=jnp.float32)
    y = jnp.maximum(acc + b_ref[0], 0.0)
    o_ref[...] = jnp.where(mask, y, o_ref[...])


def _grouped_matmul(x_sorted, W, b, gid, blk, lo, hi, maxp):
    n, din = x_sorted.shape
    e, _, dout = W.shape
    grid_spec = pltpu.PrefetchScalarGridSpec(
        num_scalar_prefetch=4,
        grid=(maxp,),
        in_specs=[
            pl.BlockSpec((_BM, din), lambda s, g, bk, l, h: (bk[s], 0)),
            pl.BlockSpec((1, din, dout), lambda s, g, bk, l, h: (g[s], 0, 0)),
            pl.BlockSpec((1, 1, dout), lambda s, g, bk, l, h: (g[s], 0, 0)),
        ],
        out_specs=pl.BlockSpec((_BM, dout), lambda s, g, bk, l, h: (bk[s], 0)),
    )
    return pl.pallas_call(
        _gmm_kernel,
        grid_spec=grid_spec,
        out_shape=jax.ShapeDtypeStruct((n, dout), jnp.float32),
    )(gid, blk, lo, hi, x_sorted, W, b.reshape(e, 1, dout))


def _tile_map(flat, e, nb, bm):
    maxp = nb + e - 1
    counts = jnp.zeros((e,), jnp.int32).at[flat].add(1)
    csum = jnp.cumsum(counts)
    offsets = csum - counts
    first = offsets // bm
    last = (offsets + counts - 1) // bm
    t = jnp.where(counts > 0, last - first + 1, 0)
    tcum = jnp.cumsum(t)
    p_total = tcum[-1]
    parange = jnp.arange(maxp, dtype=jnp.int32)
    eidx = jnp.searchsorted(tcum, parange, side="right").astype(jnp.int32)
    e_pad = jnp.searchsorted(tcum, p_total - 1, side="right").astype(jnp.int32)
    valid = parange < p_total
    eidx = jnp.where(valid, eidx, e_pad)
    k = parange - (tcum[eidx] - t[eidx])
    blk = jnp.minimum(first[eidx] + k, nb - 1)
    lo = jnp.where(valid, offsets[eidx], 0)
    hi = jnp.where(valid, offsets[eidx] + counts[eidx], 0)
    return eidx, blk, lo, hi, maxp


def kernel(input_batch, indices, W, b):
    n, _ = input_batch.shape
    e = W.shape[0]
    nb = n // _BM
    flat = indices[:, 0].astype(jnp.int32)
    order = jnp.argsort(flat, stable=True).astype(jnp.int32)
    gid, blk, lo, hi, maxp = _tile_map(flat, e, nb, _BM)
    x_sorted = _sc_gather_rows(input_batch, order, n, input_batch.shape[1])
    return _grouped_matmul(x_sorted, W, b, gid, blk, lo, hi, maxp)


# speedup vs baseline: 4.1431x; 1.0004x over previous
import functools

import jax
import jax.numpy as jnp
from jax import lax
from jax.experimental import pallas as pl
from jax.experimental.pallas import tpu as pltpu
from jax.experimental.pallas import tpu_sc as plsc

_NC = 2
_NS = 16
_NW = _NC * _NS

_BM = 128


def _sc_gather_rows(table, idx, n, d):
    b_per_w = n // _NW
    mesh = plsc.VectorSubcoreMesh(core_axis_name="c", subcore_axis_name="s")

    @functools.partial(
        pl.kernel,
        mesh=mesh,
        out_type=jax.ShapeDtypeStruct((n, d), jnp.float32),
        scratch_types=[
            pltpu.VMEM((b_per_w,), jnp.int32),
            pltpu.VMEM((b_per_w, d), jnp.float32),
            pltpu.SemaphoreType.DMA,
        ],
    )
    def gather_kernel(table_hbm, idx_hbm, out_hbm, idx_v, rows_v, sem):
        wid = lax.axis_index("s") * _NC + lax.axis_index("c")
        base = wid * b_per_w
        pltpu.sync_copy(idx_hbm.at[pl.ds(base, b_per_w)], idx_v)
        pltpu.async_copy(table_hbm.at[idx_v], rows_v, sem).wait()
        pltpu.sync_copy(rows_v, out_hbm.at[pl.ds(base, b_per_w)])

    return gather_kernel(table, idx)


def _gmm_kernel(gid_ref, blk_ref, lo_ref, hi_ref, x_ref, w_ref, b_ref, o_ref):
    s = pl.program_id(0)
    lo = lo_ref[s]
    hi = hi_ref[s]
    blk = blk_ref[s]
    bm = o_ref.shape[0]
    rows = blk * bm + lax.broadcasted_iota(jnp.int32, (bm, 1), 0)
    mask = (rows >= lo) & (rows < hi)
    acc = jnp.dot(
        x_ref[...].astype(jnp.bfloat16),
        w_ref[0].astype(jnp.bfloat16),
        preferred_element_type=jnp.float32,
    )
    y = jnp.maximum(acc + b_ref[0], 0.0)
    o_ref[...] = jnp.where(mask, y, o_ref[...])


def _grouped_matmul(x_sorted, W, b, gid, blk, lo, hi, maxp):
    n, din = x_sorted.shape
    e, _, dout = W.shape
    grid_spec = pltpu.PrefetchScalarGridSpec(
        num_scalar_prefetch=4,
        grid=(maxp,),
        in_specs=[
            pl.BlockSpec((_BM, din), lambda s, g, bk, l, h: (bk[s], 0)),
            pl.BlockSpec((1, din, dout), lambda s, g, bk, l, h: (g[s], 0, 0)),
            pl.BlockSpec((1, 1, dout), lambda s, g, bk, l, h: (g[s], 0, 0)),
        ],
        out_specs=pl.BlockSpec((_BM, dout), lambda s, g, bk, l, h: (bk[s], 0)),
    )
    return pl.pallas_call(
        _gmm_kernel,
        grid_spec=grid_spec,
        out_shape=jax.ShapeDtypeStruct((n, dout), jnp.float32),
    )(gid, blk, lo, hi, x_sorted, W, b.reshape(e, 1, dout))


def _tile_map(flat, e, nb, bm):
    maxp = nb + e - 1
    counts = jnp.zeros((e,), jnp.int32).at[flat].add(1)
    csum = jnp.cumsum(counts)
    offsets = csum - counts
    first = offsets // bm
    last = (offsets + counts - 1) // bm
    t = jnp.where(counts > 0, last - first + 1, 0)
    tcum = jnp.cumsum(t)
    p_total = tcum[-1]
    parange = jnp.arange(maxp, dtype=jnp.int32)
    eidx = jnp.searchsorted(tcum, parange, side="right").astype(jnp.int32)
    e_pad = jnp.searchsorted(tcum, p_total - 1, side="right").astype(jnp.int32)
    valid = parange < p_total
    eidx = jnp.where(valid, eidx, e_pad)
    k = parange - (tcum[eidx] - t[eidx])
    blk = jnp.minimum(first[eidx] + k, nb - 1)
    lo = jnp.where(valid, offsets[eidx], 0)
    hi = jnp.where(valid, offsets[eidx] + counts[eidx], 0)
    return eidx, blk, lo, hi, maxp


def kernel(input_batch, indices, W, b):
    n, _ = input_batch.shape
    e = W.shape[0]
    nb = n // _BM
    flat = indices[:, 0].astype(jnp.int32)
    order = jnp.argsort(flat, stable=True).astype(jnp.int32)
    gid, blk, lo, hi, maxp = _tile_map(flat, e, nb, _BM)
    x_sorted = _sc_gather_rows(input_batch, order, n, input_batch.shape[1])
    return _grouped_matmul(x_sorted, W, b, gid, blk, lo, hi, maxp)
